# baseline (speedup 1.0000x reference)
import functools

import jax
import jax.numpy as jnp
from jax import lax
from jax.experimental import pallas as pl
from jax.experimental.pallas import tpu as pltpu
from jax.experimental.pallas import tpu_sc as plsc

F = 100
V = 100
D = 128
B = 4096
LOGB = 12

R = B * F
NW = 32
RW = R // NW
CHUNK = 128
NCHUNK = RW // CHUNK
LANES = 16
NLOADF = 5
TROWS = NLOADF * V + 4


def _embed_body(xt_hbm, tbl_hbm, out_hbm, xtbuf, tblbuf, stage, s0, s1):
  ssems = (s0, s1)
  wid = lax.axis_index("c") * 16 + lax.axis_index("s")
  base = wid * RW
  f_base = lax.shift_right_logical(base, LOGB)
  f_start = pl.multiple_of(
      lax.min(jnp.bitwise_and(f_base * V, -8), F * V - TROWS), 8)

  pltpu.sync_copy(xt_hbm.at[pl.ds(base, RW)], xtbuf)
  pltpu.sync_copy(tbl_hbm.at[pl.ds(f_start, TROWS)], tblbuf.at[pl.ds(0, TROWS)])

  iota = lax.iota(jnp.int32, LANES)
  czero = iota * 0

  def construct(c, slot):
    slotv = czero + slot

    def grp(g, _):
      r0 = c * CHUNK + g * LANES
      xv = xtbuf[pl.ds(r0, LANES)]
      pos = base + r0 + iota
      fv = lax.shift_right_logical(pos, LOGB)
      liv = fv * V + xv - f_start
      riv = g * LANES + iota
      for d in range(D):
        colv = czero + d
        vals = plsc.load_gather(tblbuf, [liv, colv])
        plsc.store_scatter(stage, [slotv, riv, colv], vals)
      return 0

    lax.fori_loop(0, CHUNK // LANES, grp, 0)

  def store_desc(c, slot):
    return pltpu.make_async_copy(
        stage.at[slot], out_hbm.at[pl.ds(base + c * CHUNK, CHUNK)],
        ssems[slot])

  def pair_body(p, _):
    for slot in range(2):
      c = 2 * p + slot

      @pl.when(p >= 1)
      def _wait():
        store_desc(c - 2, slot).wait()

      construct(c, slot)
      store_desc(c, slot).start()
    return 0

  lax.fori_loop(0, NCHUNK // 2, pair_body, 0)
  for slot in range(2):
    store_desc(NCHUNK - 2 + slot, slot).wait()


@jax.jit
def _run(xt_flat, tbl_flat):
  mesh = plsc.VectorSubcoreMesh(core_axis_name="c", subcore_axis_name="s")
  k = pl.kernel(
      _embed_body,
      out_type=jax.ShapeDtypeStruct((R, D), jnp.float32),
      mesh=mesh,
      compiler_params=pltpu.CompilerParams(needs_layout_passes=False),
      scratch_types=[
          pltpu.VMEM((RW,), jnp.int32),
          pltpu.VMEM((TROWS + 12, D), jnp.float32),
          pltpu.VMEM((2, CHUNK, D), jnp.float32),
          pltpu.SemaphoreType.DMA,
          pltpu.SemaphoreType.DMA,
      ],
  )
  return k(xt_flat, tbl_flat)


def kernel(x, tables):
  xt_flat = x.T.reshape(R)
  tbl_flat = tables.reshape(F * V, D)
  out = _run(xt_flat, tbl_flat)
  return out.reshape(F, B, D).transpose(1, 0, 2)


# speedup vs baseline: 15.2506x; 15.2506x over previous
import functools

import jax
import jax.numpy as jnp
from jax import lax
from jax.experimental import pallas as pl
from jax.experimental.pallas import tpu as pltpu
from jax.experimental.pallas import tpu_sc as plsc

F = 100
V = 100
D = 128
B = 4096
LOGB = 12

R = B * F
NW = 32
RW = R // NW
CHUNK = 128
NCHUNK = RW // CHUNK
LANES = 16
NBUF = 2
LAG = 1


def _gather_body(xt_hbm, tbl_hbm, out_hbm, idxbuf, rows, shared_tbl, *sems):
  gsems = sems[:NBUF]
  ssems = sems[NBUF:]
  sid = lax.axis_index("s")
  wid = lax.axis_index("c") * 16 + sid
  base = wid * RW

  @pl.when(sid == 0)
  def _stage_table():
    pltpu.sync_copy(tbl_hbm, shared_tbl)

  pltpu.sync_copy(xt_hbm.at[pl.ds(base, RW)], idxbuf)

  iota = lax.iota(jnp.int32, LANES)

  def idx_step(j, _):
    xv = idxbuf[pl.ds(j * LANES, LANES)]
    pos = base + j * LANES + iota
    fv = lax.shift_right_logical(pos, LOGB)
    idxbuf[pl.ds(j * LANES, LANES)] = fv * V + xv
    return 0

  lax.fori_loop(0, RW // LANES, idx_step, 0)
  plsc.subcore_barrier()

  def gfire(c):
    b = c % NBUF
    return pltpu.async_copy(
        shared_tbl.at[idxbuf.at[pl.ds(c * CHUNK, CHUNK)]], rows.at[b],
        gsems[b])

  def sfire(c):
    b = c % NBUF
    return pltpu.async_copy(
        rows.at[b], out_hbm.at[pl.ds(base + c * CHUNK, CHUNK)], ssems[b])

  gdesc = {}
  sdesc = {}
  for c in range(NBUF):
    gdesc[c] = gfire(c)
  for c in range(NCHUNK):
    gdesc.pop(c).wait()
    sdesc[c] = sfire(c)
    c2 = c - LAG
    if c2 >= 0 and c2 + NBUF < NCHUNK:
      sdesc.pop(c2).wait()
      gdesc[c2 + NBUF] = gfire(c2 + NBUF)
  for c in sorted(sdesc):
    sdesc.pop(c).wait()


@jax.jit
def _run(xt_flat, tbl_flat):
  mesh = plsc.VectorSubcoreMesh(core_axis_name="c", subcore_axis_name="s")
  k = pl.kernel(
      _gather_body,
      out_type=jax.ShapeDtypeStruct((R, D), jnp.float32),
      mesh=mesh,
      scratch_types=[
          pltpu.VMEM((RW,), jnp.int32),
          pltpu.VMEM((NBUF, CHUNK, D), jnp.float32),
          pltpu.VMEM_SHARED((F * V, D), jnp.float32),
      ] + [pltpu.SemaphoreType.DMA] * (2 * NBUF),
  )
  return k(xt_flat, tbl_flat)


def kernel(x, tables):
  xt_flat = x.T.reshape(R)
  tbl_flat = tables.reshape(F * V, D)
  out = _run(xt_flat, tbl_flat)
  return out.reshape(F, B, D).transpose(1, 0, 2)


# speedup vs baseline: 16.1014x; 1.0558x over previous
import functools

import jax
import jax.numpy as jnp
from jax import lax
from jax.experimental import pallas as pl
from jax.experimental.pallas import tpu as pltpu
from jax.experimental.pallas import tpu_sc as plsc

F = 100
V = 100
D = 128
B = 4096
LOGB = 12

R = B * F
NW = 32
RW = R // NW
CHUNK = 64
NCHUNK = RW // CHUNK
LANES = 16
NBUF = 4
LAG = 2


def _gather_body(xt_hbm, tbl_hbm, out_hbm, idxbuf, rows, shared_tbl, *sems):
  gsems = sems[:NBUF]
  ssems = sems[NBUF:]
  sid = lax.axis_index("s")
  wid = lax.axis_index("c") * 16 + sid
  base = wid * RW

  @pl.when(sid == 0)
  def _stage_table():
    pltpu.sync_copy(tbl_hbm, shared_tbl)

  pltpu.sync_copy(xt_hbm.at[pl.ds(base, RW)], idxbuf)

  iota = lax.iota(jnp.int32, LANES)

  def idx_step(j, _):
    xv = idxbuf[pl.ds(j * LANES, LANES)]
    pos = base + j * LANES + iota
    fv = lax.shift_right_logical(pos, LOGB)
    idxbuf[pl.ds(j * LANES, LANES)] = fv * V + xv
    return 0

  lax.fori_loop(0, RW // LANES, idx_step, 0)
  plsc.subcore_barrier()

  def gfire(c):
    b = c % NBUF
    return pltpu.async_copy(
        shared_tbl.at[idxbuf.at[pl.ds(c * CHUNK, CHUNK)]], rows.at[b],
        gsems[b])

  def sfire(c):
    b = c % NBUF
    return pltpu.async_copy(
        rows.at[b], out_hbm.at[pl.ds(base + c * CHUNK, CHUNK)], ssems[b])

  gdesc = {}
  sdesc = {}
  for c in range(NBUF):
    gdesc[c] = gfire(c)
  for c in range(NCHUNK):
    gdesc.pop(c).wait()
    sdesc[c] = sfire(c)
    c2 = c - LAG
    if c2 >= 0 and c2 + NBUF < NCHUNK:
      sdesc.pop(c2).wait()
      gdesc[c2 + NBUF] = gfire(c2 + NBUF)
  for c in sorted(sdesc):
    sdesc.pop(c).wait()


@jax.jit
def _run(xt_flat, tbl_flat):
  mesh = plsc.VectorSubcoreMesh(core_axis_name="c", subcore_axis_name="s")
  k = pl.kernel(
      _gather_body,
      out_type=jax.ShapeDtypeStruct((R, D), jnp.float32),
      mesh=mesh,
      scratch_types=[
          pltpu.VMEM((RW,), jnp.int32),
          pltpu.VMEM((NBUF, CHUNK, D), jnp.float32),
          pltpu.VMEM_SHARED((F * V, D), jnp.float32),
      ] + [pltpu.SemaphoreType.DMA] * (2 * NBUF),
  )
  return k(xt_flat, tbl_flat)


def kernel(x, tables):
  xt_flat = x.T.reshape(R)
  tbl_flat = tables.reshape(F * V, D)
  out = _run(xt_flat, tbl_flat)
  return out.reshape(F, B, D).transpose(1, 0, 2)
